# baseline (speedup 1.0000x reference)
import jax
import jax.numpy as jnp
from jax import lax
from jax.experimental import pallas as pl
from jax.experimental.pallas import tpu as pltpu

_BB = 2


def _csra_body(xa_ref, xb_ref, ct_ref, cw_ref, cb_ref, fw_ref, fb_ref,
               lam_ref, out_ref):
    bb, s, dh = xa_ref.shape
    c = cw_ref.shape[0]
    i = pl.program_id(0)
    xa = xa_ref[...].reshape(bb * s, dh)
    xb = xb_ref[...].reshape(bb * s, dh)
    cw_aug = jnp.concatenate(
        [cw_ref[...], jnp.ones((1, 2 * dh), jnp.float32)]
    ).astype(jnp.bfloat16)
    raw = (
        lax.dot_general(xa.astype(jnp.bfloat16), cw_aug[:, :dh],
                        (((1,), (1,)), ((), ())),
                        preferred_element_type=jnp.float32)
        + lax.dot_general(xb.astype(jnp.bfloat16), cw_aug[:, dh:],
                          (((1,), (1,)), ((), ())),
                          preferred_element_type=jnp.float32))
    attn = jax.nn.sigmoid(raw[:, :c] + cb_ref[...])
    rowsum = raw[:, c:c + 1]
    s_attn = jnp.sum((attn * rowsum).reshape(bb, s, -1), axis=1) \
        * (1.0 / (s * 2 * dh))
    group = out_ref.shape[0] // bb
    ct = ct_ref[pl.ds((i // group) * (group * bb), group * bb), :]
    s_global = lax.dot_general(
        ct, fw_ref[...],
        (((1,), (1,)), ((), ())),
        preferred_element_type=jnp.float32) + fb_ref[...]
    s_attn2 = jnp.concatenate([s_attn] * group, axis=0)
    rows = lax.broadcasted_iota(jnp.int32, (group * bb, c), 0)
    mask = (rows // bb) == (i % group)
    result = s_global + lam_ref[0, 0] * s_attn2
    out_ref[...] = jnp.where(mask, result, out_ref[...])


def kernel(patch_tokens, class_token, conv_w, conv_b, fc_w, fc_b, lam):
    b, s, d = patch_tokens.shape
    c = conv_w.shape[0]
    cb2 = conv_b.reshape(1, c)
    fb2 = fc_b.reshape(1, c)
    lam2 = jnp.asarray(lam, jnp.float32).reshape(1, 1)

    xspec = lambda half: pl.BlockSpec(
        (_BB, s, d // 2), lambda i, _h=half: (i, 0, _h))
    return pl.pallas_call(
        _csra_body,
        grid=(b // _BB,),
        in_specs=[
            xspec(0),
            xspec(1),
            pl.BlockSpec((b, d), lambda i: (0, 0)),
            pl.BlockSpec((c, d), lambda i: (0, 0)),
            pl.BlockSpec((1, c), lambda i: (0, 0)),
            pl.BlockSpec((c, d), lambda i: (0, 0)),
            pl.BlockSpec((1, c), lambda i: (0, 0)),
            pl.BlockSpec((1, 1), lambda i: (0, 0)),
        ],
        out_specs=pl.BlockSpec((8, c), lambda i: (i // (8 // _BB), 0)),
        out_shape=jax.ShapeDtypeStruct((b, c), jnp.float32),
        compiler_params=pltpu.CompilerParams(
            dimension_semantics=("arbitrary",),
            vmem_limit_bytes=100 * 1024 * 1024),
    )(patch_tokens, patch_tokens, class_token, conv_w, cb2, fc_w, fb2, lam2)


# speedup vs baseline: 1.2042x; 1.2042x over previous
import jax
import jax.numpy as jnp
from jax import lax
from jax.experimental import pallas as pl
from jax.experimental.pallas import tpu as pltpu

_BB = 4


def _csra_body(xa_ref, xb_ref, ct_ref, cw_ref, cb_ref, fw_ref, fb_ref,
               lam_ref, out_ref):
    bb, s, dh = xa_ref.shape
    c = cw_ref.shape[0]
    i = pl.program_id(0)
    xa = xa_ref[...].reshape(bb * s, dh)
    xb = xb_ref[...].reshape(bb * s, dh)
    cw_aug = jnp.concatenate(
        [cw_ref[...], jnp.ones((1, 2 * dh), jnp.float32)]
    ).astype(jnp.bfloat16)
    raw = (
        lax.dot_general(xa.astype(jnp.bfloat16), cw_aug[:, :dh],
                        (((1,), (1,)), ((), ())),
                        preferred_element_type=jnp.float32)
        + lax.dot_general(xb.astype(jnp.bfloat16), cw_aug[:, dh:],
                          (((1,), (1,)), ((), ())),
                          preferred_element_type=jnp.float32))
    attn = jax.nn.sigmoid(raw[:, :c] + cb_ref[...])
    rowsum = raw[:, c:c + 1]
    s_attn = jnp.sum((attn * rowsum).reshape(bb, s, -1), axis=1) \
        * (1.0 / (s * 2 * dh))
    group = out_ref.shape[0] // bb
    ct = ct_ref[pl.ds((i // group) * (group * bb), group * bb), :]
    s_global = lax.dot_general(
        ct, fw_ref[...],
        (((1,), (1,)), ((), ())),
        preferred_element_type=jnp.float32) + fb_ref[...]
    s_attn2 = jnp.concatenate([s_attn] * group, axis=0)
    rows = lax.broadcasted_iota(jnp.int32, (group * bb, c), 0)
    mask = (rows // bb) == (i % group)
    result = s_global + lam_ref[0, 0] * s_attn2
    out_ref[...] = jnp.where(mask, result, out_ref[...])


def kernel(patch_tokens, class_token, conv_w, conv_b, fc_w, fc_b, lam):
    b, s, d = patch_tokens.shape
    c = conv_w.shape[0]
    cb2 = conv_b.reshape(1, c)
    fb2 = fc_b.reshape(1, c)
    lam2 = jnp.asarray(lam, jnp.float32).reshape(1, 1)

    xspec = lambda half: pl.BlockSpec(
        (_BB, s, d // 2), lambda i, _h=half: (i, 0, _h))
    return pl.pallas_call(
        _csra_body,
        grid=(b // _BB,),
        in_specs=[
            xspec(0),
            xspec(1),
            pl.BlockSpec((b, d), lambda i: (0, 0)),
            pl.BlockSpec((c, d), lambda i: (0, 0)),
            pl.BlockSpec((1, c), lambda i: (0, 0)),
            pl.BlockSpec((c, d), lambda i: (0, 0)),
            pl.BlockSpec((1, c), lambda i: (0, 0)),
            pl.BlockSpec((1, 1), lambda i: (0, 0)),
        ],
        out_specs=pl.BlockSpec((8, c), lambda i: (i // (8 // _BB), 0)),
        out_shape=jax.ShapeDtypeStruct((b, c), jnp.float32),
        compiler_params=pltpu.CompilerParams(
            dimension_semantics=("arbitrary",),
            vmem_limit_bytes=100 * 1024 * 1024),
    )(patch_tokens, patch_tokens, class_token, conv_w, cb2, fc_w, fb2, lam2)
